# baseline (speedup 1.0000x reference)
import functools

import jax
import jax.numpy as jnp
from jax import lax
from jax.experimental import pallas as pl
from jax.experimental.pallas import tpu as pltpu
from jax.experimental.pallas import tpu_sc as plsc

_NC = 2
_NS = 16
_NW = _NC * _NS
_CHUNK = 88200


def kernel(samples, sample_rate):
    del sample_rate
    shape = samples.shape
    flat = samples.reshape(-1)
    n = flat.shape[0]
    per_w = n // _NW
    n_chunks = per_w // _CHUNK
    assert per_w * _NW == n and n_chunks * _CHUNK == per_w

    mesh = plsc.VectorSubcoreMesh(core_axis_name="c", subcore_axis_name="s")

    @functools.partial(
        pl.kernel,
        mesh=mesh,
        out_type=jax.ShapeDtypeStruct((n,), jnp.float32),
        scratch_types=[pltpu.VMEM((_CHUNK,), jnp.float32)],
    )
    def sc_copy(in_hbm, out_hbm, buf):
        wid = lax.axis_index("s") * _NC + lax.axis_index("c")
        base = wid * per_w

        def body(i, carry):
            off = base + i * _CHUNK
            pltpu.sync_copy(in_hbm.at[pl.ds(off, _CHUNK)], buf)
            pltpu.sync_copy(buf, out_hbm.at[pl.ds(off, _CHUNK)])
            return carry

        lax.fori_loop(0, n_chunks, body, 0)

    return sc_copy(flat).reshape(shape)


# speedup vs baseline: 56.5365x; 56.5365x over previous
import jax
import jax.numpy as jnp
from jax.experimental import pallas as pl

_BLOCK_ROWS = 8
_BLOCK_COLS = 441000


def _copy_body(in_ref, out_ref):
    out_ref[...] = in_ref[...]


def kernel(samples, sample_rate):
    del sample_rate
    rows, cols = samples.shape
    grid = (pl.cdiv(rows, _BLOCK_ROWS), pl.cdiv(cols, _BLOCK_COLS))
    return pl.pallas_call(
        _copy_body,
        out_shape=jax.ShapeDtypeStruct(samples.shape, samples.dtype),
        grid=grid,
        in_specs=[pl.BlockSpec((_BLOCK_ROWS, _BLOCK_COLS), lambda i, j: (i, j))],
        out_specs=pl.BlockSpec((_BLOCK_ROWS, _BLOCK_COLS), lambda i, j: (i, j)),
    )(samples)
